# baseline (speedup 1.0000x reference)
import functools
import math

import jax
import jax.numpy as jnp
from jax import lax
from jax.experimental import pallas as pl
from jax.experimental.pallas import tpu as pltpu
from jax.experimental.pallas import tpu_sc as plsc

D_MODEL = 512
SCALE = math.sqrt(D_MODEL)

NC = 2
NS = 16
NW = NC * NS
LANES = 16

CHUNK = 16
NBUF = 2


def _make_gather_kernel(B: int):
    assert B % (8 * NW) == 0
    b_per_w = B // NW
    assert b_per_w % (NBUF * CHUNK) == 0
    n_iters = b_per_w // (NBUF * CHUNK)

    mesh = plsc.VectorSubcoreMesh(core_axis_name="c", subcore_axis_name="s")

    @functools.partial(
        pl.kernel,
        mesh=mesh,
        out_type=jax.ShapeDtypeStruct((B, D_MODEL), jnp.float32),
        scratch_types=(
            [pltpu.VMEM((b_per_w,), jnp.int32)]
            + [pltpu.VMEM((CHUNK, D_MODEL), jnp.float32)] * (2 * NBUF)
            + [pltpu.SemaphoreType.DMA] * (2 * NBUF)
        ),
    )
    def gather_scale(table_hbm, idx_hbm, out_hbm, idx_all, *bufs_and_sems):
        ibuf = bufs_and_sems[0:NBUF]
        obuf = bufs_and_sems[NBUF:2 * NBUF]
        sg = bufs_and_sems[2 * NBUF:3 * NBUF]
        ss = bufs_and_sems[3 * NBUF:4 * NBUF]
        wid = lax.axis_index("s") * NC + lax.axis_index("c")
        w_base = wid * b_per_w

        pltpu.sync_copy(idx_hbm.at[pl.ds(pl.multiple_of(w_base, 8), b_per_w)],
                        idx_all)

        def start_gather(b, g):
            off = pl.multiple_of(g * CHUNK, 8)
            pltpu.async_copy(table_hbm.at[idx_all.at[pl.ds(off, CHUNK)]],
                             ibuf[b], sg[b])

        def wait_gather(b, g):
            off = pl.multiple_of(g * CHUNK, 8)
            pltpu.make_async_copy(table_hbm.at[idx_all.at[pl.ds(off, CHUNK)]],
                                  ibuf[b], sg[b]).wait()

        def scale_rows(b):
            def row_body(r, c2):
                for i in range(D_MODEL // LANES):
                    sl = pl.ds(i * LANES, LANES)
                    obuf[b][r, sl] = ibuf[b][r, sl] * SCALE
                return c2

            lax.fori_loop(0, CHUNK, row_body, 0)

        def start_scatter(b, g):
            base = pl.multiple_of(w_base + g * CHUNK, 8)
            pltpu.async_copy(obuf[b], out_hbm.at[pl.ds(base, CHUNK)], ss[b])

        def wait_scatter(b, g):
            base = pl.multiple_of(w_base + g * CHUNK, 8)
            pltpu.make_async_copy(obuf[b], out_hbm.at[pl.ds(base, CHUNK)], ss[b]).wait()

        for b in range(NBUF):
            start_gather(b, b)

        def iter_body(k, carry):
            g0 = NBUF * k
            for b in range(NBUF):
                g = g0 + b
                wait_gather(b, g)

                @pl.when(k > 0)
                def _(b=b, g=g):
                    wait_scatter(b, g - NBUF)

                scale_rows(b)
                start_scatter(b, g)

                @pl.when(k < n_iters - 1)
                def _(b=b, g=g):
                    start_gather(b, g + NBUF)
            return carry

        lax.fori_loop(0, n_iters, iter_body, 0)
        for b in range(NBUF):
            wait_scatter(b, NBUF * (n_iters - 1) + b)

    return gather_scale


def kernel(x, table):
    B = x.size
    flat_idx = x.T.reshape((B,)).astype(jnp.int32)
    out = _make_gather_kernel(B)(table, flat_idx)
    out3 = out.reshape((x.shape[1], x.shape[0], D_MODEL))
    return out3.transpose(1, 0, 2)


# speedup vs baseline: 1.1332x; 1.1332x over previous
import functools
import math

import jax
import jax.numpy as jnp
from jax import lax
from jax.experimental import pallas as pl
from jax.experimental.pallas import tpu as pltpu
from jax.experimental.pallas import tpu_sc as plsc

D_MODEL = 512
SCALE = math.sqrt(D_MODEL)

NC = 2
NS = 16
NW = NC * NS
LANES = 16

CHUNK = 32
NBUF = 2


def _make_gather_kernel(B: int):
    assert B % (8 * NW) == 0
    b_per_w = B // NW
    assert b_per_w % (NBUF * CHUNK) == 0
    n_iters = b_per_w // (NBUF * CHUNK)

    mesh = plsc.VectorSubcoreMesh(core_axis_name="c", subcore_axis_name="s")

    @functools.partial(
        pl.kernel,
        mesh=mesh,
        out_type=jax.ShapeDtypeStruct((B, D_MODEL), jnp.float32),
        scratch_types=(
            [pltpu.VMEM((b_per_w,), jnp.int32)]
            + [pltpu.VMEM((CHUNK, D_MODEL), jnp.float32)] * (2 * NBUF)
            + [pltpu.VMEM_SHARED((NS, NBUF, CHUNK, D_MODEL), jnp.float32)]
            + [pltpu.SemaphoreType.DMA] * (2 * NBUF)
        ),
    )
    def gather_scale(table_hbm, idx_hbm, out_hbm, idx_all, *bufs_and_sems):
        ibuf = bufs_and_sems[0:NBUF]
        obuf = bufs_and_sems[NBUF:2 * NBUF]
        shared = bufs_and_sems[2 * NBUF]
        sg = bufs_and_sems[2 * NBUF + 1:3 * NBUF + 1]
        ss = bufs_and_sems[3 * NBUF + 1:4 * NBUF + 1]
        sid = lax.axis_index("s")
        wid = sid * NC + lax.axis_index("c")
        w_base = wid * b_per_w

        pltpu.sync_copy(idx_hbm.at[pl.ds(pl.multiple_of(w_base, 8), b_per_w)],
                        idx_all)

        def start_gather(b, g):
            off = pl.multiple_of(g * CHUNK, 8)
            pltpu.async_copy(table_hbm.at[idx_all.at[pl.ds(off, CHUNK)]],
                             ibuf[b], sg[b])

        def wait_gather(b, g):
            off = pl.multiple_of(g * CHUNK, 8)
            pltpu.make_async_copy(table_hbm.at[idx_all.at[pl.ds(off, CHUNK)]],
                                  ibuf[b], sg[b]).wait()

        def scale_rows(b):
            def row_body(r, c2):
                for i in range(D_MODEL // LANES):
                    sl = pl.ds(i * LANES, LANES)
                    obuf[b][r, sl] = ibuf[b][r, sl] * SCALE
                return c2

            lax.fori_loop(0, CHUNK, row_body, 0)

        def start_scatter(b, g):
            base = pl.multiple_of(w_base + g * CHUNK, 8)
            pltpu.sync_copy(obuf[b], shared.at[sid, b])
            pltpu.async_copy(shared.at[sid, b], out_hbm.at[pl.ds(base, CHUNK)],
                             ss[b])

        def wait_scatter(b, g):
            base = pl.multiple_of(w_base + g * CHUNK, 8)
            pltpu.make_async_copy(shared.at[sid, b],
                                  out_hbm.at[pl.ds(base, CHUNK)], ss[b]).wait()

        for b in range(NBUF):
            start_gather(b, b)

        def iter_body(k, carry):
            g0 = NBUF * k
            for b in range(NBUF):
                g = g0 + b
                wait_gather(b, g)

                @pl.when(k > 0)
                def _(b=b, g=g):
                    wait_scatter(b, g - NBUF)

                scale_rows(b)
                start_scatter(b, g)

                @pl.when(k < n_iters - 1)
                def _(b=b, g=g):
                    start_gather(b, g + NBUF)
            return carry

        lax.fori_loop(0, n_iters, iter_body, 0)
        for b in range(NBUF):
            wait_scatter(b, NBUF * (n_iters - 1) + b)

    return gather_scale


def kernel(x, table):
    B = x.size
    flat_idx = x.T.reshape((B,)).astype(jnp.int32)
    out = _make_gather_kernel(B)(table, flat_idx)
    out3 = out.reshape((x.shape[1], x.shape[0], D_MODEL))
    return out3.transpose(1, 0, 2)


# speedup vs baseline: 1.1670x; 1.0298x over previous
import functools
import math

import jax
import jax.numpy as jnp
from jax import lax
from jax.experimental import pallas as pl
from jax.experimental.pallas import tpu as pltpu
from jax.experimental.pallas import tpu_sc as plsc

D_MODEL = 512
SCALE = math.sqrt(D_MODEL)

NC = 2
NS = 16
NW = NC * NS
LANES = 16

CHUNK = 32
NBUF = 2


def _make_gather_kernel(B: int):
    assert B % (8 * NW) == 0
    b_per_w = B // NW
    assert b_per_w % (NBUF * CHUNK) == 0
    n_iters = b_per_w // (NBUF * CHUNK)

    mesh = plsc.VectorSubcoreMesh(core_axis_name="c", subcore_axis_name="s")

    @functools.partial(
        pl.kernel,
        mesh=mesh,
        out_type=jax.ShapeDtypeStruct((B, D_MODEL), jnp.float32),
        scratch_types=(
            [pltpu.VMEM((b_per_w,), jnp.int32)]
            + [pltpu.VMEM((CHUNK, D_MODEL), jnp.float32)] * (2 * NBUF)
            + [pltpu.SemaphoreType.DMA] * (2 * NBUF)
        ),
    )
    def gather_scale(table_hbm, idx_hbm, out_hbm, idx_all, *bufs_and_sems):
        ibuf = bufs_and_sems[0:NBUF]
        obuf = bufs_and_sems[NBUF:2 * NBUF]
        sg = bufs_and_sems[2 * NBUF:3 * NBUF]
        ss = bufs_and_sems[3 * NBUF:4 * NBUF]
        wid = lax.axis_index("s") * NC + lax.axis_index("c")
        w_base = wid * b_per_w

        pltpu.sync_copy(idx_hbm.at[pl.ds(pl.multiple_of(w_base, 8), b_per_w)],
                        idx_all)

        def start_gather(b, g):
            off = pl.multiple_of(g * CHUNK, 8)
            pltpu.async_copy(table_hbm.at[idx_all.at[pl.ds(off, CHUNK)]],
                             ibuf[b], sg[b])

        def wait_gather(b, g):
            off = pl.multiple_of(g * CHUNK, 8)
            pltpu.make_async_copy(table_hbm.at[idx_all.at[pl.ds(off, CHUNK)]],
                                  ibuf[b], sg[b]).wait()

        def scale_rows(b):
            def row_body(r, c2):
                for i in range(D_MODEL // LANES):
                    sl = pl.ds(i * LANES, LANES)
                    obuf[b][r, sl] = ibuf[b][r, sl] * SCALE
                return c2

            lax.fori_loop(0, CHUNK, row_body, 0)

        def start_scatter(b, g):
            base = pl.multiple_of(w_base + g * CHUNK, 8)
            pltpu.async_copy(obuf[b], out_hbm.at[pl.ds(base, CHUNK)], ss[b])

        def wait_scatter(b, g):
            base = pl.multiple_of(w_base + g * CHUNK, 8)
            pltpu.make_async_copy(obuf[b], out_hbm.at[pl.ds(base, CHUNK)], ss[b]).wait()

        for b in range(NBUF):
            start_gather(b, b)

        def iter_body(k, carry):
            g0 = NBUF * k
            for b in range(NBUF):
                g = g0 + b
                wait_gather(b, g)

                @pl.when(k > 0)
                def _(b=b, g=g):
                    wait_scatter(b, g - NBUF)

                scale_rows(b)
                start_scatter(b, g)

                @pl.when(k < n_iters - 1)
                def _(b=b, g=g):
                    start_gather(b, g + NBUF)
            return carry

        lax.fori_loop(0, n_iters, iter_body, 0)
        for b in range(NBUF):
            wait_scatter(b, NBUF * (n_iters - 1) + b)

    return gather_scale


def kernel(x, table):
    B = x.size
    flat_idx = x.T.reshape((B,)).astype(jnp.int32)
    out = _make_gather_kernel(B)(table, flat_idx)
    out3 = out.reshape((x.shape[1], x.shape[0], D_MODEL))
    return out3.transpose(1, 0, 2)
